# Initial kernel scaffold; baseline (speedup 1.0000x reference)
import jax
import jax.numpy as jnp
from jax.experimental import pallas as pl


def kernel(x, w_g):
    raise NotImplementedError("write your pallas kernel here")


# speedup vs baseline: 1.0340x; 1.0340x over previous
import functools

import jax
import jax.numpy as jnp
from jax import lax
from jax.experimental import pallas as pl
from jax.experimental.pallas import tpu as pltpu

B, T, N, D = 2, 2048, 24, 1024
K = 12
KEEP = N - K
TB = 128


def _body(x_ref, wg_ref, out_ref):
    xt = x_ref[...]
    wg = wg_ref[...]
    scores = lax.dot_general(
        xt.reshape(TB * N, D), wg.reshape(D, 1),
        dimension_numbers=(((1,), (0,)), ((), ())),
        preferred_element_type=jnp.float32,
    ).reshape(TB, N)

    si = scores[:, :, None]
    sj = scores[:, None, :]
    ii = lax.broadcasted_iota(jnp.int32, (TB, N, N), 1)
    jj = lax.broadcasted_iota(jnp.int32, (TB, N, N), 2)
    beats = (sj > si) | ((sj == si) & (jj < ii))
    rank = jnp.sum(beats.astype(jnp.int32), axis=2)
    keep = rank < KEEP

    neg = jnp.float32(-1e30)
    masked = jnp.where(keep, scores, neg)
    m = jnp.max(masked, axis=-1, keepdims=True)
    e = jnp.where(keep, jnp.exp(scores - m), 0.0)
    w = e / jnp.sum(e, axis=-1, keepdims=True)

    out_ref[...] = jnp.sum(w[:, :, None] * xt, axis=1)


@jax.jit
def kernel(x, w_g):
    F = B * T
    xf = x.reshape(F, N, D)
    wg2 = w_g.reshape(1, D)
    out = pl.pallas_call(
        _body,
        grid=(F // TB,),
        in_specs=[
            pl.BlockSpec((TB, N, D), lambda i: (i, 0, 0)),
            pl.BlockSpec((1, D), lambda i: (0, 0)),
        ],
        out_specs=pl.BlockSpec((TB, D), lambda i: (i, 0)),
        out_shape=jax.ShapeDtypeStruct((F, D), jnp.float32),
    )(xf, wg2)
    return out.reshape(B, T, D)


# speedup vs baseline: 3.7215x; 3.5992x over previous
import functools

import jax
import jax.numpy as jnp
from jax import lax
from jax.experimental import pallas as pl
from jax.experimental.pallas import tpu as pltpu

B, T, N, D = 2, 2048, 24, 1024
K = 12
KEEP = N - K
TB = 128


def _body(x_ref, wg_ref, out_ref):
    xt = x_ref[...]
    wg = wg_ref[...]
    scores = lax.dot_general(
        xt.reshape(TB * N, D), wg.reshape(D, 1),
        dimension_numbers=(((1,), (0,)), ((), ())),
        preferred_element_type=jnp.float32,
    ).reshape(TB, N)

    st = scores.T
    si = st[:, None, :]
    sj = st[None, :, :]
    ii = lax.broadcasted_iota(jnp.int32, (N, N, TB), 0)
    jj = lax.broadcasted_iota(jnp.int32, (N, N, TB), 1)
    beats = (sj > si) | ((sj == si) & (jj < ii))
    rank = jnp.sum(beats.astype(jnp.int32), axis=1)
    keep = rank < KEEP

    neg = jnp.float32(-1e30)
    masked = jnp.where(keep, st, neg)
    m = jnp.max(masked, axis=0, keepdims=True)
    e = jnp.where(keep, jnp.exp(st - m), 0.0)
    w = (e / jnp.sum(e, axis=0, keepdims=True)).T

    out_ref[...] = jnp.sum(w[:, :, None] * xt, axis=1)


@jax.jit
def kernel(x, w_g):
    F = B * T
    xf = x.reshape(F, N, D)
    wg2 = w_g.reshape(1, D)
    out = pl.pallas_call(
        _body,
        grid=(F // TB,),
        in_specs=[
            pl.BlockSpec((TB, N, D), lambda i: (i, 0, 0)),
            pl.BlockSpec((1, D), lambda i: (0, 0)),
        ],
        out_specs=pl.BlockSpec((TB, D), lambda i: (i, 0)),
        out_shape=jax.ShapeDtypeStruct((F, D), jnp.float32),
    )(xf, wg2)
    return out.reshape(B, T, D)
